# Initial kernel scaffold; baseline (speedup 1.0000x reference)
import jax
import jax.numpy as jnp
from jax.experimental import pallas as pl


def kernel(id_batch, neigh_idx_drug, neigh_idx_gene, neigh_idx_cell, drug_features, gene_features, cell_features, W_drug, b_drug, W_gene, b_gene, W_cell, b_cell, att):
    raise NotImplementedError("write your pallas kernel here")


# speedup vs baseline: 1.9121x; 1.9121x over previous
import functools

import jax
import jax.numpy as jnp
from jax import lax
from jax.experimental import pallas as pl
from jax.experimental.pallas import tpu as pltpu
from jax.experimental.pallas import tpu_sc as plsc

NW = 32
LANES = 16
D = 128


def _project(x, w, b, bn=512):
    n, din = x.shape

    def body(x_ref, w_ref, b_ref, o_ref):
        o_ref[...] = (
            jnp.dot(x_ref[...], w_ref[...], preferred_element_type=jnp.float32)
            + b_ref[0:1, :]
        )

    return pl.pallas_call(
        body,
        grid=(pl.cdiv(n, bn),),
        in_specs=[
            pl.BlockSpec((bn, din), lambda i: (i, 0)),
            pl.BlockSpec((din, D), lambda i: (0, 0)),
            pl.BlockSpec((8, D), lambda i: (0, 0)),
        ],
        out_specs=pl.BlockSpec((bn, D), lambda i: (i, 0)),
        out_shape=jax.ShapeDtypeStruct((n, D), jnp.float32),
    )(x, w, jnp.pad(b.reshape(1, D), ((0, 7), (0, 0))))


def _make_sc_gather_mean(k_fan, rows_pw, chunk_b):
    idxw = chunk_b * k_fan
    chunks = rows_pw // chunk_b
    schunks = rows_pw // idxw
    b_pad = NW * rows_pw
    mesh = plsc.VectorSubcoreMesh(core_axis_name="c", subcore_axis_name="s")
    out_sds = jax.ShapeDtypeStruct((b_pad, D), jnp.float32)

    @functools.partial(
        pl.kernel,
        out_type=(out_sds, out_sds, out_sds, out_sds),
        mesh=mesh,
        scratch_types=[
            pltpu.VMEM((chunks, idxw), jnp.int32),
            pltpu.VMEM((schunks, idxw), jnp.int32),
            pltpu.VMEM((idxw, D), jnp.float32),
            pltpu.VMEM((rows_pw, D), jnp.float32),
            pltpu.SemaphoreType.DMA,
        ],
    )
    def sc_kernel(emb_d, emb_g, emb_c, idx_d, idx_g, idx_c, idx_s,
                  out_d, out_g, out_c, out_s,
                  idx_v, idx_sv, rows_v, out_buf, sem):
        w = lax.axis_index("s") * 2 + lax.axis_index("c")
        base = w * rows_pw
        inv_k = jnp.float32(1.0 / k_fan)

        for tbl, idx_hbm, out_hbm in (
            (emb_d, idx_d, out_d),
            (emb_g, idx_g, out_g),
            (emb_c, idx_c, out_c),
        ):
            pltpu.sync_copy(idx_hbm.at[w], idx_v)

            def chunk_body(c, carry, tbl=tbl):
                pltpu.async_copy(tbl.at[idx_v.at[c]], rows_v, sem).wait()
                for r in range(chunk_b):
                    orow = c * chunk_b + r
                    for g in range(D // LANES):
                        sl = pl.ds(g * LANES, LANES)
                        acc = rows_v[r * k_fan, sl]
                        for k in range(1, k_fan):
                            acc = acc + rows_v[r * k_fan + k, sl]
                        out_buf[orow, sl] = acc * inv_k
                return carry

            lax.fori_loop(0, chunks, chunk_body, 0)
            pltpu.sync_copy(out_buf, out_hbm.at[pl.ds(base, rows_pw)])

        pltpu.sync_copy(idx_s.at[w], idx_sv)
        for c in range(schunks):
            pltpu.async_copy(
                emb_d.at[idx_sv.at[c]], out_buf.at[pl.ds(c * idxw, idxw)], sem
            ).wait()
        pltpu.sync_copy(out_buf, out_s.at[pl.ds(base, rows_pw)])

    return sc_kernel


def _combine(s, d, g, c, att, n_out, bn=512):
    def body(s_ref, d_ref, g_ref, c_ref, a_ref, o_ref):
        sv = s_ref[...]
        dv = d_ref[...]
        gv = g_ref[...]
        cv = c_ref[...]
        a1 = a_ref[0:1, :]
        a2 = a_ref[1:2, :]
        s_half = jnp.sum(sv * a1, axis=1, keepdims=True)

        def score(x):
            t = s_half + jnp.sum(x * a2, axis=1, keepdims=True)
            return jnp.where(t >= 0, t, 0.2 * t)

        t0, t1, t2, t3 = score(sv), score(dv), score(gv), score(cv)
        m = jnp.maximum(jnp.maximum(t0, t1), jnp.maximum(t2, t3))
        e0 = jnp.exp(t0 - m)
        e1 = jnp.exp(t1 - m)
        e2 = jnp.exp(t2 - m)
        e3 = jnp.exp(t3 - m)
        denom = e0 + e1 + e2 + e3
        o_ref[...] = (e0 * sv + e1 * dv + e2 * gv + e3 * cv) / denom

    spec = pl.BlockSpec((bn, D), lambda i: (i, 0))
    return pl.pallas_call(
        body,
        grid=(pl.cdiv(n_out, bn),),
        in_specs=[spec, spec, spec, spec,
                  pl.BlockSpec((8, D), lambda i: (0, 0))],
        out_specs=spec,
        out_shape=jax.ShapeDtypeStruct((n_out, D), jnp.float32),
    )(s, d, g, c, jnp.pad(att.reshape(2, D), ((0, 6), (0, 0))))


def kernel(id_batch, neigh_idx_drug, neigh_idx_gene, neigh_idx_cell,
           drug_features, gene_features, cell_features,
           W_drug, b_drug, W_gene, b_gene, W_cell, b_cell, att):
    b = id_batch.shape[0]
    k_fan = neigh_idx_drug.shape[1]
    chunk_b = 8
    rows_pw = pl.cdiv(b, NW * chunk_b) * chunk_b
    b_pad = NW * rows_pw
    idxw = chunk_b * k_fan

    emb_d = _project(drug_features, W_drug, b_drug)
    emb_g = _project(gene_features, W_gene, b_gene)
    emb_c = _project(cell_features, W_cell, b_cell)

    def prep(idx):
        p = jnp.pad(idx.astype(jnp.int32), ((0, b_pad - b), (0, 0)))
        return p.reshape(NW, rows_pw // chunk_b, idxw)

    idx_d = prep(neigh_idx_drug)
    idx_g = prep(neigh_idx_gene)
    idx_c = prep(neigh_idx_cell)
    idx_s = jnp.pad(id_batch.astype(jnp.int32), (0, b_pad - b)).reshape(
        NW, rows_pw // idxw, idxw)

    sc = _make_sc_gather_mean(k_fan, rows_pw, chunk_b)
    agg_d, agg_g, agg_c, self_emb = sc(
        emb_d, emb_g, emb_c, idx_d, idx_g, idx_c, idx_s)

    return _combine(self_emb, agg_d, agg_g, agg_c, att, b)


# speedup vs baseline: 2.2684x; 1.1863x over previous
import functools

import jax
import jax.numpy as jnp
from jax import lax
from jax.experimental import pallas as pl
from jax.experimental.pallas import tpu as pltpu
from jax.experimental.pallas import tpu_sc as plsc

NW = 32
LANES = 16
D = 128


def _project(x, w, b, bn=512):
    n, din = x.shape

    def body(x_ref, w_ref, b_ref, o_ref):
        o_ref[...] = (
            jnp.dot(x_ref[...], w_ref[...], preferred_element_type=jnp.float32)
            + b_ref[0:1, :]
        )

    return pl.pallas_call(
        body,
        grid=(pl.cdiv(n, bn),),
        in_specs=[
            pl.BlockSpec((bn, din), lambda i: (i, 0)),
            pl.BlockSpec((din, D), lambda i: (0, 0)),
            pl.BlockSpec((8, D), lambda i: (0, 0)),
        ],
        out_specs=pl.BlockSpec((bn, D), lambda i: (i, 0)),
        out_shape=jax.ShapeDtypeStruct((n, D), jnp.float32),
    )(x, w, jnp.pad(b.reshape(1, D), ((0, 7), (0, 0))))


def _make_sc_gather_mean(k_fan, rows_pw, chunk_b):
    idxw = chunk_b * k_fan
    chunks = rows_pw // chunk_b
    schunks = rows_pw // idxw
    b_pad = NW * rows_pw
    mesh = plsc.VectorSubcoreMesh(core_axis_name="c", subcore_axis_name="s")
    out_sds = jax.ShapeDtypeStruct((b_pad, D), jnp.float32)

    @functools.partial(
        pl.kernel,
        out_type=(out_sds, out_sds, out_sds, out_sds),
        mesh=mesh,
        scratch_types=[
            pltpu.VMEM((chunks, idxw), jnp.int32),
            pltpu.VMEM((schunks, idxw), jnp.int32),
            pltpu.VMEM((2, idxw, D), jnp.float32),
            pltpu.VMEM((rows_pw, D), jnp.float32),
            pltpu.SemaphoreType.DMA((2,)),
        ],
    )
    def sc_kernel(emb_d, emb_g, emb_c, idx_d, idx_g, idx_c, idx_s,
                  out_d, out_g, out_c, out_s,
                  idx_v, idx_sv, rows_v, out_buf, sem):
        w = lax.axis_index("s") * 2 + lax.axis_index("c")
        base = w * rows_pw
        inv_k = jnp.float32(1.0 / k_fan)

        for tbl, idx_hbm, out_hbm in (
            (emb_d, idx_d, out_d),
            (emb_g, idx_g, out_g),
            (emb_c, idx_c, out_c),
        ):
            pltpu.sync_copy(idx_hbm.at[w], idx_v)
            pltpu.async_copy(tbl.at[idx_v.at[0]], rows_v.at[0], sem.at[0])

            def chunk_body(c, carry, tbl=tbl):
                p = lax.rem(c, 2)
                q = lax.rem(c + 1, 2)
                pltpu.make_async_copy(
                    tbl.at[idx_v.at[c]], rows_v.at[p], sem.at[p]).wait()

                @pl.when(c + 1 < chunks)
                def _():
                    pltpu.async_copy(
                        tbl.at[idx_v.at[c + 1]], rows_v.at[q], sem.at[q])

                for r in range(chunk_b):
                    orow = c * chunk_b + r
                    for g in range(D // LANES):
                        sl = pl.ds(g * LANES, LANES)
                        acc = rows_v[p, r * k_fan, sl]
                        for k in range(1, k_fan):
                            acc = acc + rows_v[p, r * k_fan + k, sl]
                        out_buf[orow, sl] = acc * inv_k
                return carry

            lax.fori_loop(0, chunks, chunk_body, 0)
            pltpu.sync_copy(out_buf, out_hbm.at[pl.ds(base, rows_pw)])

        pltpu.sync_copy(idx_s.at[w], idx_sv)
        copies = [
            pltpu.async_copy(
                emb_d.at[idx_sv.at[c]], out_buf.at[pl.ds(c * idxw, idxw)],
                sem.at[0])
            for c in range(schunks)
        ]
        for cp in copies:
            cp.wait()
        pltpu.sync_copy(out_buf, out_s.at[pl.ds(base, rows_pw)])

    return sc_kernel


def _combine(s, d, g, c, att, n_out, bn=512):
    def body(s_ref, d_ref, g_ref, c_ref, a_ref, o_ref):
        sv = s_ref[...]
        dv = d_ref[...]
        gv = g_ref[...]
        cv = c_ref[...]
        a1 = a_ref[0:1, :]
        a2 = a_ref[1:2, :]
        s_half = jnp.sum(sv * a1, axis=1, keepdims=True)

        def score(x):
            t = s_half + jnp.sum(x * a2, axis=1, keepdims=True)
            return jnp.where(t >= 0, t, 0.2 * t)

        t0, t1, t2, t3 = score(sv), score(dv), score(gv), score(cv)
        m = jnp.maximum(jnp.maximum(t0, t1), jnp.maximum(t2, t3))
        e0 = jnp.exp(t0 - m)
        e1 = jnp.exp(t1 - m)
        e2 = jnp.exp(t2 - m)
        e3 = jnp.exp(t3 - m)
        denom = e0 + e1 + e2 + e3
        o_ref[...] = (e0 * sv + e1 * dv + e2 * gv + e3 * cv) / denom

    spec = pl.BlockSpec((bn, D), lambda i: (i, 0))
    return pl.pallas_call(
        body,
        grid=(pl.cdiv(n_out, bn),),
        in_specs=[spec, spec, spec, spec,
                  pl.BlockSpec((8, D), lambda i: (0, 0))],
        out_specs=spec,
        out_shape=jax.ShapeDtypeStruct((n_out, D), jnp.float32),
    )(s, d, g, c, jnp.pad(att.reshape(2, D), ((0, 6), (0, 0))))


def kernel(id_batch, neigh_idx_drug, neigh_idx_gene, neigh_idx_cell,
           drug_features, gene_features, cell_features,
           W_drug, b_drug, W_gene, b_gene, W_cell, b_cell, att):
    b = id_batch.shape[0]
    k_fan = neigh_idx_drug.shape[1]
    chunk_b = 8
    rows_pw = pl.cdiv(b, NW * chunk_b) * chunk_b
    b_pad = NW * rows_pw
    idxw = chunk_b * k_fan

    emb_d = _project(drug_features, W_drug, b_drug)
    emb_g = _project(gene_features, W_gene, b_gene)
    emb_c = _project(cell_features, W_cell, b_cell)

    def prep(idx):
        p = jnp.pad(idx.astype(jnp.int32), ((0, b_pad - b), (0, 0)))
        return p.reshape(NW, rows_pw // chunk_b, idxw)

    idx_d = prep(neigh_idx_drug)
    idx_g = prep(neigh_idx_gene)
    idx_c = prep(neigh_idx_cell)
    idx_s = jnp.pad(id_batch.astype(jnp.int32), (0, b_pad - b)).reshape(
        NW, rows_pw // idxw, idxw)

    sc = _make_sc_gather_mean(k_fan, rows_pw, chunk_b)
    agg_d, agg_g, agg_c, self_emb = sc(
        emb_d, emb_g, emb_c, idx_d, idx_g, idx_c, idx_s)

    return _combine(self_emb, agg_d, agg_g, agg_c, att, b)


# speedup vs baseline: 2.4017x; 1.0588x over previous
import functools

import jax
import jax.numpy as jnp
from jax import lax
from jax.experimental import pallas as pl
from jax.experimental.pallas import tpu as pltpu
from jax.experimental.pallas import tpu_sc as plsc

NW = 32
LANES = 16
D = 128


def _project(x, w, b, bn=512):
    n, din = x.shape

    def body(x_ref, w_ref, b_ref, o_ref):
        o_ref[...] = (
            jnp.dot(x_ref[...], w_ref[...], preferred_element_type=jnp.float32)
            + b_ref[0:1, :]
        )

    return pl.pallas_call(
        body,
        grid=(pl.cdiv(n, bn),),
        in_specs=[
            pl.BlockSpec((bn, din), lambda i: (i, 0)),
            pl.BlockSpec((din, D), lambda i: (0, 0)),
            pl.BlockSpec((8, D), lambda i: (0, 0)),
        ],
        out_specs=pl.BlockSpec((bn, D), lambda i: (i, 0)),
        out_shape=jax.ShapeDtypeStruct((n, D), jnp.float32),
    )(x, w, jnp.pad(b.reshape(1, D), ((0, 7), (0, 0))))


def _make_sc_gather_mean(k_fan, rows_pw, chunk_b):
    idxw = chunk_b * k_fan
    chunks = rows_pw // chunk_b
    schunks = rows_pw // idxw
    b_pad = NW * rows_pw
    mesh = plsc.VectorSubcoreMesh(core_axis_name="c", subcore_axis_name="s")
    out_sds = jax.ShapeDtypeStruct((b_pad, D), jnp.float32)

    @functools.partial(
        pl.kernel,
        out_type=(out_sds, out_sds, out_sds, out_sds),
        mesh=mesh,
        scratch_types=[
            pltpu.VMEM((chunks, idxw), jnp.int32),
            pltpu.VMEM((schunks, idxw), jnp.int32),
            pltpu.VMEM((4, idxw, D), jnp.float32),
            pltpu.VMEM((rows_pw, D), jnp.float32),
            pltpu.SemaphoreType.DMA((4,)),
        ],
    )
    def sc_kernel(emb_d, emb_g, emb_c, idx_d, idx_g, idx_c, idx_s,
                  out_d, out_g, out_c, out_s,
                  idx_v, idx_sv, rows_v, out_buf, sem):
        w = lax.axis_index("s") * 2 + lax.axis_index("c")
        base = w * rows_pw
        inv_k = jnp.float32(1.0 / k_fan)

        for tbl, idx_hbm, out_hbm in (
            (emb_d, idx_d, out_d),
            (emb_g, idx_g, out_g),
            (emb_c, idx_c, out_c),
        ):
            pltpu.sync_copy(idx_hbm.at[w], idx_v)
            for c0 in range(3):
                pltpu.async_copy(
                    tbl.at[idx_v.at[c0]], rows_v.at[c0], sem.at[c0])

            def chunk_body(c, carry, tbl=tbl):
                p = lax.rem(c, 4)
                q = lax.rem(c + 3, 4)
                pltpu.make_async_copy(
                    tbl.at[idx_v.at[c]], rows_v.at[p], sem.at[p]).wait()

                @pl.when(c + 3 < chunks)
                def _():
                    pltpu.async_copy(
                        tbl.at[idx_v.at[c + 3]], rows_v.at[q], sem.at[q])

                for r in range(chunk_b):
                    orow = c * chunk_b + r
                    for g in range(D // LANES):
                        sl = pl.ds(g * LANES, LANES)
                        vals = [rows_v[p, r * k_fan + k, sl]
                                for k in range(k_fan)]
                        while len(vals) > 1:
                            vals = [a + b for a, b in
                                    zip(vals[::2], vals[1::2])] + (
                                        [vals[-1]] if len(vals) % 2 else [])
                        out_buf[orow, sl] = vals[0] * inv_k
                return carry

            lax.fori_loop(0, chunks, chunk_body, 0)
            pltpu.sync_copy(out_buf, out_hbm.at[pl.ds(base, rows_pw)])

        pltpu.sync_copy(idx_s.at[w], idx_sv)
        copies = [
            pltpu.async_copy(
                emb_d.at[idx_sv.at[c]], out_buf.at[pl.ds(c * idxw, idxw)],
                sem.at[0])
            for c in range(schunks)
        ]
        for cp in copies:
            cp.wait()
        pltpu.sync_copy(out_buf, out_s.at[pl.ds(base, rows_pw)])

    return sc_kernel


def _combine(s, d, g, c, att, n_out, bn=512):
    def body(s_ref, d_ref, g_ref, c_ref, a_ref, o_ref):
        sv = s_ref[...]
        dv = d_ref[...]
        gv = g_ref[...]
        cv = c_ref[...]
        a1 = a_ref[0:1, :]
        a2 = a_ref[1:2, :]
        s_half = jnp.sum(sv * a1, axis=1, keepdims=True)

        def score(x):
            t = s_half + jnp.sum(x * a2, axis=1, keepdims=True)
            return jnp.where(t >= 0, t, 0.2 * t)

        t0, t1, t2, t3 = score(sv), score(dv), score(gv), score(cv)
        m = jnp.maximum(jnp.maximum(t0, t1), jnp.maximum(t2, t3))
        e0 = jnp.exp(t0 - m)
        e1 = jnp.exp(t1 - m)
        e2 = jnp.exp(t2 - m)
        e3 = jnp.exp(t3 - m)
        denom = e0 + e1 + e2 + e3
        o_ref[...] = (e0 * sv + e1 * dv + e2 * gv + e3 * cv) / denom

    spec = pl.BlockSpec((bn, D), lambda i: (i, 0))
    return pl.pallas_call(
        body,
        grid=(pl.cdiv(n_out, bn),),
        in_specs=[spec, spec, spec, spec,
                  pl.BlockSpec((8, D), lambda i: (0, 0))],
        out_specs=spec,
        out_shape=jax.ShapeDtypeStruct((n_out, D), jnp.float32),
    )(s, d, g, c, jnp.pad(att.reshape(2, D), ((0, 6), (0, 0))))


def kernel(id_batch, neigh_idx_drug, neigh_idx_gene, neigh_idx_cell,
           drug_features, gene_features, cell_features,
           W_drug, b_drug, W_gene, b_gene, W_cell, b_cell, att):
    b = id_batch.shape[0]
    k_fan = neigh_idx_drug.shape[1]
    chunk_b = 8
    rows_pw = pl.cdiv(b, NW * chunk_b) * chunk_b
    b_pad = NW * rows_pw
    idxw = chunk_b * k_fan

    emb_d = _project(drug_features, W_drug, b_drug)
    emb_g = _project(gene_features, W_gene, b_gene)
    emb_c = _project(cell_features, W_cell, b_cell)

    def prep(idx):
        p = jnp.pad(idx.astype(jnp.int32), ((0, b_pad - b), (0, 0)))
        return p.reshape(NW, rows_pw // chunk_b, idxw)

    idx_d = prep(neigh_idx_drug)
    idx_g = prep(neigh_idx_gene)
    idx_c = prep(neigh_idx_cell)
    idx_s = jnp.pad(id_batch.astype(jnp.int32), (0, b_pad - b)).reshape(
        NW, rows_pw // idxw, idxw)

    sc = _make_sc_gather_mean(k_fan, rows_pw, chunk_b)
    agg_d, agg_g, agg_c, self_emb = sc(
        emb_d, emb_g, emb_c, idx_d, idx_g, idx_c, idx_s)

    return _combine(self_emb, agg_d, agg_g, agg_c, att, b)


# speedup vs baseline: 2.4576x; 1.0233x over previous
import functools

import jax
import jax.numpy as jnp
from jax import lax
from jax.experimental import pallas as pl
from jax.experimental.pallas import tpu as pltpu
from jax.experimental.pallas import tpu_sc as plsc

NW = 32
NS = 16
LANES = 16
D = 128
CHUNK_B = 8
SELF_B = 16
C0_ROWS = 464
C1_ROWS = 176


def _project(x, w, b, bn=512):
    n, din = x.shape

    def body(x_ref, w_ref, b_ref, o_ref):
        o_ref[...] = (
            jnp.dot(x_ref[...], w_ref[...], preferred_element_type=jnp.float32)
            + b_ref[0:1, :]
        )

    return pl.pallas_call(
        body,
        grid=(pl.cdiv(n, bn),),
        in_specs=[
            pl.BlockSpec((bn, din), lambda i: (i, 0)),
            pl.BlockSpec((din, D), lambda i: (0, 0)),
            pl.BlockSpec((8, D), lambda i: (0, 0)),
        ],
        out_specs=pl.BlockSpec((bn, D), lambda i: (i, 0)),
        out_shape=jax.ShapeDtypeStruct((n, D), jnp.float32),
    )(x, w, jnp.pad(b.reshape(1, D), ((0, 7), (0, 0))))


def _make_sc_gather_mean(k_fan, b_pad):
    idxw = CHUNK_B * k_fan
    nch0 = C0_ROWS // CHUNK_B
    nch1 = C1_ROWS // CHUNK_B
    nself0 = C0_ROWS // SELF_B
    nself1 = C1_ROWS // SELF_B
    c0_total = NS * C0_ROWS
    mesh = plsc.VectorSubcoreMesh(core_axis_name="c", subcore_axis_name="s")
    out_sds = jax.ShapeDtypeStruct((b_pad, D), jnp.float32)

    @functools.partial(
        pl.kernel,
        out_type=(out_sds, out_sds, out_sds, out_sds),
        mesh=mesh,
        scratch_types=[
            pltpu.VMEM((nch0 * idxw,), jnp.int32),
            pltpu.VMEM((nself0 * SELF_B,), jnp.int32),
            pltpu.VMEM((4, idxw, D), jnp.float32),
            pltpu.VMEM((C0_ROWS, D), jnp.float32),
            pltpu.SemaphoreType.DMA((4,)),
            pltpu.SemaphoreType.DMA,
        ],
    )
    def sc_kernel(emb_d, emb_g, emb_c, idx_d, idx_g, idx_c, idx_s,
                  out_d, out_g, out_c, out_s,
                  idx_v, idx_sv, rows_v, pool_buf, sem, sem_out):
        c = lax.axis_index("c")
        s = lax.axis_index("s")
        is0 = c == 0
        nchunks = jnp.where(is0, nch0, nch1)
        nself = jnp.where(is0, nself0, nself1)
        row_base = pl.multiple_of(
            jnp.where(is0, s * C0_ROWS, c0_total + s * C1_ROWS), 16)
        idx_base = pl.multiple_of(row_base * k_fan, 16)
        inv_k = jnp.float32(1.0 / k_fan)

        def tree(vs):
            while len(vs) > 1:
                vs = [a + b for a, b in zip(vs[::2], vs[1::2])] + (
                    [vs[-1]] if len(vs) % 2 else [])
            return vs[0]

        for tbl, idx_hbm, out_hbm in (
            (emb_d, idx_d, out_d),
            (emb_g, idx_g, out_g),
            (emb_c, idx_c, out_c),
        ):
            pltpu.sync_copy(
                idx_hbm.at[pl.ds(idx_base, nch0 * idxw)], idx_v)

            def idx_at(ci):
                return idx_v.at[pl.ds(pl.multiple_of(ci * idxw, 16), idxw)]

            for c0 in range(3):
                pltpu.async_copy(
                    tbl.at[idx_at(c0)], rows_v.at[c0], sem.at[c0])

            def chunk_body(ci, carry, tbl=tbl, out_hbm=out_hbm, idx_at=idx_at):
                p = lax.rem(ci, 4)
                q = lax.rem(ci + 3, 4)
                pltpu.make_async_copy(
                    tbl.at[idx_at(ci)], rows_v.at[p], sem.at[p]).wait()

                @pl.when(ci + 3 < nchunks)
                def _():
                    pltpu.async_copy(
                        tbl.at[idx_at(ci + 3)], rows_v.at[q], sem.at[q])

                for r in range(CHUNK_B):
                    orow = ci * CHUNK_B + r
                    for g in range(D // LANES):
                        sl = pl.ds(g * LANES, LANES)
                        pool_buf[orow, sl] = tree(
                            [rows_v[p, r * k_fan + k, sl]
                             for k in range(k_fan)]) * inv_k
                pltpu.async_copy(
                    pool_buf.at[pl.ds(ci * CHUNK_B, CHUNK_B)],
                    out_hbm.at[pl.ds(
                        pl.multiple_of(row_base + ci * CHUNK_B, CHUNK_B),
                        CHUNK_B)],
                    sem_out)

                @pl.when(ci >= 6)
                def _():
                    pltpu.make_async_copy(
                        pool_buf.at[pl.ds(0, CHUNK_B)],
                        out_hbm.at[pl.ds(row_base, CHUNK_B)],
                        sem_out).wait()
                return carry

            lax.fori_loop(0, nchunks, chunk_body, 0)

            def drain_body(_, carry, out_hbm=out_hbm):
                pltpu.make_async_copy(
                    pool_buf.at[pl.ds(0, CHUNK_B)],
                    out_hbm.at[pl.ds(row_base, CHUNK_B)],
                    sem_out).wait()
                return carry

            lax.fori_loop(0, jnp.minimum(nchunks, 6), drain_body, 0)

        pltpu.sync_copy(
            idx_s.at[pl.ds(row_base, nself0 * SELF_B)], idx_sv)

        def fire_write(j):
            pltpu.async_copy(
                pool_buf.at[pl.ds(j * SELF_B, SELF_B)],
                out_s.at[pl.ds(
                    pl.multiple_of(row_base + j * SELF_B, SELF_B), SELF_B)],
                sem_out)

        def wait_gather(j):
            pltpu.make_async_copy(
                emb_d.at[idx_sv.at[pl.ds(
                    pl.multiple_of(j * SELF_B, SELF_B), SELF_B)]],
                pool_buf.at[pl.ds(j * SELF_B, SELF_B)],
                sem.at[lax.rem(j, 4)]).wait()

        def self_body(j, carry):
            pltpu.async_copy(
                emb_d.at[idx_sv.at[pl.ds(
                    pl.multiple_of(j * SELF_B, SELF_B), SELF_B)]],
                pool_buf.at[pl.ds(j * SELF_B, SELF_B)],
                sem.at[lax.rem(j, 4)])

            @pl.when(j >= 3)
            def _():
                wait_gather(j - 3)
                fire_write(j - 3)

            @pl.when(j >= 9)
            def _():
                pltpu.make_async_copy(
                    pool_buf.at[pl.ds(0, SELF_B)],
                    out_s.at[pl.ds(row_base, SELF_B)],
                    sem_out).wait()
            return carry

        lax.fori_loop(0, nself, self_body, 0)

        def self_tail(t, carry):
            j = nself - 3 + t
            wait_gather(j)
            fire_write(j)
            return carry

        lax.fori_loop(0, 3, self_tail, 0)
        for _ in range(9):
            pltpu.make_async_copy(
                pool_buf.at[pl.ds(0, SELF_B)],
                out_s.at[pl.ds(row_base, SELF_B)],
                sem_out).wait()

    return sc_kernel


def _combine(s, d, g, c, att, n_out, bn=512):
    def body(s_ref, d_ref, g_ref, c_ref, a_ref, o_ref):
        sv = s_ref[...]
        dv = d_ref[...]
        gv = g_ref[...]
        cv = c_ref[...]
        a1 = a_ref[0:1, :]
        a2 = a_ref[1:2, :]
        s_half = jnp.sum(sv * a1, axis=1, keepdims=True)

        def score(x):
            t = s_half + jnp.sum(x * a2, axis=1, keepdims=True)
            return jnp.where(t >= 0, t, 0.2 * t)

        t0, t1, t2, t3 = score(sv), score(dv), score(gv), score(cv)
        m = jnp.maximum(jnp.maximum(t0, t1), jnp.maximum(t2, t3))
        e0 = jnp.exp(t0 - m)
        e1 = jnp.exp(t1 - m)
        e2 = jnp.exp(t2 - m)
        e3 = jnp.exp(t3 - m)
        denom = e0 + e1 + e2 + e3
        o_ref[...] = (e0 * sv + e1 * dv + e2 * gv + e3 * cv) / denom

    spec = pl.BlockSpec((bn, D), lambda i: (i, 0))
    return pl.pallas_call(
        body,
        grid=(pl.cdiv(n_out, bn),),
        in_specs=[spec, spec, spec, spec,
                  pl.BlockSpec((8, D), lambda i: (0, 0))],
        out_specs=spec,
        out_shape=jax.ShapeDtypeStruct((n_out, D), jnp.float32),
    )(s, d, g, c, jnp.pad(att.reshape(2, D), ((0, 6), (0, 0))))


def kernel(id_batch, neigh_idx_drug, neigh_idx_gene, neigh_idx_cell,
           drug_features, gene_features, cell_features,
           W_drug, b_drug, W_gene, b_gene, W_cell, b_cell, att):
    b = id_batch.shape[0]
    k_fan = neigh_idx_drug.shape[1]
    b_pad = NS * (C0_ROWS + C1_ROWS)
    idxw = CHUNK_B * k_fan
    nrows = b_pad // CHUNK_B
    nch_pad = NS * (C0_ROWS // CHUNK_B) + (NS - 1) * (
        C1_ROWS // CHUNK_B) + C0_ROWS // CHUNK_B
    nself_pad = NS * C0_ROWS + (NS - 1) * C1_ROWS + C0_ROWS

    emb_d = _project(drug_features, W_drug, b_drug)
    emb_g = _project(gene_features, W_gene, b_gene)
    emb_c = _project(cell_features, W_cell, b_cell)

    def prep(idx):
        p = jnp.pad(idx.astype(jnp.int32), ((0, b_pad - b), (0, 0)))
        return jnp.pad(p.reshape(-1), (0, (nch_pad - nrows) * idxw))

    idx_d = prep(neigh_idx_drug)
    idx_g = prep(neigh_idx_gene)
    idx_c = prep(neigh_idx_cell)
    idx_s = jnp.pad(id_batch.astype(jnp.int32), (0, nself_pad - b))

    sc = _make_sc_gather_mean(k_fan, b_pad)
    agg_d, agg_g, agg_c, self_emb = sc(
        emb_d, emb_g, emb_c, idx_d, idx_g, idx_c, idx_s)

    return _combine(self_emb, agg_d, agg_g, agg_c, att, b)


# speedup vs baseline: 2.5544x; 1.0394x over previous
import functools

import jax
import jax.numpy as jnp
from jax import lax
from jax.experimental import pallas as pl
from jax.experimental.pallas import tpu as pltpu
from jax.experimental.pallas import tpu_sc as plsc

NW = 32
NS = 16
LANES = 16
D = 128
CHUNK_B = 8
SELF_B = 16
C0_ROWS = 464
C1_ROWS = 176
NBUF = 8
NPOOL = 16


def _project(x, w, b, bn=512):
    n, din = x.shape

    def body(x_ref, w_ref, b_ref, o_ref):
        o_ref[...] = (
            jnp.dot(x_ref[...].astype(jnp.bfloat16),
                    w_ref[...].astype(jnp.bfloat16),
                    preferred_element_type=jnp.float32)
            + b_ref[0:1, :]
        )

    return pl.pallas_call(
        body,
        grid=(pl.cdiv(n, bn),),
        in_specs=[
            pl.BlockSpec((bn, din), lambda i: (i, 0)),
            pl.BlockSpec((din, D), lambda i: (0, 0)),
            pl.BlockSpec((8, D), lambda i: (0, 0)),
        ],
        out_specs=pl.BlockSpec((bn, D), lambda i: (i, 0)),
        out_shape=jax.ShapeDtypeStruct((n, D), jnp.float32),
    )(x, w, jnp.pad(b.reshape(1, D), ((0, 7), (0, 0))))


def _make_sc_gather_mean(k_fan, b_pad):
    idxw = CHUNK_B * k_fan
    nch0 = C0_ROWS // CHUNK_B
    nch1 = C1_ROWS // CHUNK_B
    nself0 = C0_ROWS // SELF_B
    nself1 = C1_ROWS // SELF_B
    c0_total = NS * C0_ROWS
    mesh = plsc.VectorSubcoreMesh(core_axis_name="c", subcore_axis_name="s")
    out_sds = jax.ShapeDtypeStruct((b_pad, D), jnp.float32)

    @functools.partial(
        pl.kernel,
        out_type=(out_sds, out_sds, out_sds, out_sds),
        mesh=mesh,
        scratch_types=[
            pltpu.VMEM((nch0 * idxw,), jnp.int32),
            pltpu.VMEM((nself0 * SELF_B,), jnp.int32),
            pltpu.VMEM((NBUF, idxw, D), jnp.float32),
            pltpu.VMEM((NPOOL * CHUNK_B, D), jnp.float32),
            pltpu.SemaphoreType.DMA((NBUF,)),
            pltpu.SemaphoreType.DMA,
        ],
    )
    def sc_kernel(emb_d, emb_g, emb_c, idx_d, idx_g, idx_c, idx_s,
                  out_d, out_g, out_c, out_s,
                  idx_v, idx_sv, rows_v, pool_buf, sem, sem_out):
        c = lax.axis_index("c")
        s = lax.axis_index("s")
        is0 = c == 0
        nchunks = jnp.where(is0, nch0, nch1)
        nself = jnp.where(is0, nself0, nself1)
        row_base = pl.multiple_of(
            jnp.where(is0, s * C0_ROWS, c0_total + s * C1_ROWS), 16)
        idx_base = pl.multiple_of(row_base * k_fan, 16)
        inv_k = jnp.float32(1.0 / k_fan)

        def tree(vs):
            while len(vs) > 1:
                vs = [a + b for a, b in zip(vs[::2], vs[1::2])] + (
                    [vs[-1]] if len(vs) % 2 else [])
            return vs[0]

        for tbl, idx_hbm, out_hbm in (
            (emb_d, idx_d, out_d),
            (emb_g, idx_g, out_g),
            (emb_c, idx_c, out_c),
        ):
            pltpu.sync_copy(
                idx_hbm.at[pl.ds(idx_base, nch0 * idxw)], idx_v)

            def idx_at(ci):
                return idx_v.at[pl.ds(pl.multiple_of(ci * idxw, 16), idxw)]

            for c0 in range(NBUF - 1):
                pltpu.async_copy(
                    tbl.at[idx_at(c0)], rows_v.at[c0], sem.at[c0])

            def chunk_body(ci, carry, tbl=tbl, out_hbm=out_hbm, idx_at=idx_at):
                p = lax.rem(ci, NBUF)
                q = lax.rem(ci + NBUF - 1, NBUF)
                o = lax.rem(ci, NPOOL)
                pltpu.make_async_copy(
                    tbl.at[idx_at(ci)], rows_v.at[p], sem.at[p]).wait()

                @pl.when(ci + NBUF - 1 < nchunks)
                def _():
                    pltpu.async_copy(
                        tbl.at[idx_at(ci + NBUF - 1)], rows_v.at[q],
                        sem.at[q])

                @pl.when(ci >= NPOOL)
                def _():
                    pltpu.make_async_copy(
                        pool_buf.at[pl.ds(0, CHUNK_B)],
                        out_hbm.at[pl.ds(row_base, CHUNK_B)],
                        sem_out).wait()

                for r in range(CHUNK_B):
                    orow = o * CHUNK_B + r
                    for g in range(D // LANES):
                        sl = pl.ds(g * LANES, LANES)
                        pool_buf[orow, sl] = tree(
                            [rows_v[p, r * k_fan + k, sl]
                             for k in range(k_fan)]) * inv_k
                pltpu.async_copy(
                    pool_buf.at[pl.ds(
                        pl.multiple_of(o * CHUNK_B, CHUNK_B), CHUNK_B)],
                    out_hbm.at[pl.ds(
                        pl.multiple_of(row_base + ci * CHUNK_B, CHUNK_B),
                        CHUNK_B)],
                    sem_out)
                return carry

            lax.fori_loop(0, nchunks, chunk_body, 0)

            def drain_body(_, carry, out_hbm=out_hbm):
                pltpu.make_async_copy(
                    pool_buf.at[pl.ds(0, CHUNK_B)],
                    out_hbm.at[pl.ds(row_base, CHUNK_B)],
                    sem_out).wait()
                return carry

            lax.fori_loop(0, jnp.minimum(nchunks, NPOOL), drain_body, 0)

        pltpu.sync_copy(
            idx_s.at[pl.ds(row_base, nself0 * SELF_B)], idx_sv)

        nslot = NPOOL * CHUNK_B // SELF_B

        def self_slot(j):
            return pl.ds(pl.multiple_of(lax.rem(j, nslot) * SELF_B, SELF_B),
                         SELF_B)

        def fire_write(j):
            pltpu.async_copy(
                pool_buf.at[self_slot(j)],
                out_s.at[pl.ds(
                    pl.multiple_of(row_base + j * SELF_B, SELF_B), SELF_B)],
                sem_out)

        def wait_gather(j):
            pltpu.make_async_copy(
                emb_d.at[idx_sv.at[pl.ds(
                    pl.multiple_of(j * SELF_B, SELF_B), SELF_B)]],
                pool_buf.at[self_slot(j)],
                sem.at[lax.rem(j, NBUF)]).wait()

        def self_body(j, carry):
            @pl.when(j >= nslot)
            def _():
                pltpu.make_async_copy(
                    pool_buf.at[pl.ds(0, SELF_B)],
                    out_s.at[pl.ds(row_base, SELF_B)],
                    sem_out).wait()

            pltpu.async_copy(
                emb_d.at[idx_sv.at[pl.ds(
                    pl.multiple_of(j * SELF_B, SELF_B), SELF_B)]],
                pool_buf.at[self_slot(j)],
                sem.at[lax.rem(j, NBUF)])

            @pl.when(j >= 3)
            def _():
                wait_gather(j - 3)
                fire_write(j - 3)
            return carry

        lax.fori_loop(0, nself, self_body, 0)

        def self_tail(t, carry):
            j = nself - 3 + t
            wait_gather(j)
            fire_write(j)
            return carry

        lax.fori_loop(0, 3, self_tail, 0)
        def self_drain(_, carry):
            pltpu.make_async_copy(
                pool_buf.at[pl.ds(0, SELF_B)],
                out_s.at[pl.ds(row_base, SELF_B)],
                sem_out).wait()
            return carry

        lax.fori_loop(0, jnp.minimum(nself, nslot), self_drain, 0)

    return sc_kernel


def _combine(s, d, g, c, att, n_out, bn=512):
    def body(s_ref, d_ref, g_ref, c_ref, a_ref, o_ref):
        sv = s_ref[...]
        dv = d_ref[...]
        gv = g_ref[...]
        cv = c_ref[...]
        a1 = a_ref[0:1, :]
        a2 = a_ref[1:2, :]
        s_half = jnp.sum(sv * a1, axis=1, keepdims=True)

        def score(x):
            t = s_half + jnp.sum(x * a2, axis=1, keepdims=True)
            return jnp.where(t >= 0, t, 0.2 * t)

        t0, t1, t2, t3 = score(sv), score(dv), score(gv), score(cv)
        m = jnp.maximum(jnp.maximum(t0, t1), jnp.maximum(t2, t3))
        e0 = jnp.exp(t0 - m)
        e1 = jnp.exp(t1 - m)
        e2 = jnp.exp(t2 - m)
        e3 = jnp.exp(t3 - m)
        denom = e0 + e1 + e2 + e3
        o_ref[...] = (e0 * sv + e1 * dv + e2 * gv + e3 * cv) / denom

    spec = pl.BlockSpec((bn, D), lambda i: (i, 0))
    return pl.pallas_call(
        body,
        grid=(pl.cdiv(n_out, bn),),
        in_specs=[spec, spec, spec, spec,
                  pl.BlockSpec((8, D), lambda i: (0, 0))],
        out_specs=spec,
        out_shape=jax.ShapeDtypeStruct((n_out, D), jnp.float32),
    )(s, d, g, c, jnp.pad(att.reshape(2, D), ((0, 6), (0, 0))))


def kernel(id_batch, neigh_idx_drug, neigh_idx_gene, neigh_idx_cell,
           drug_features, gene_features, cell_features,
           W_drug, b_drug, W_gene, b_gene, W_cell, b_cell, att):
    b = id_batch.shape[0]
    k_fan = neigh_idx_drug.shape[1]
    b_pad = NS * (C0_ROWS + C1_ROWS)
    idxw = CHUNK_B * k_fan
    nrows = b_pad // CHUNK_B
    nch_pad = NS * (C0_ROWS // CHUNK_B) + (NS - 1) * (
        C1_ROWS // CHUNK_B) + C0_ROWS // CHUNK_B
    nself_pad = NS * C0_ROWS + (NS - 1) * C1_ROWS + C0_ROWS

    emb_d = _project(drug_features, W_drug, b_drug)
    emb_g = _project(gene_features, W_gene, b_gene)
    emb_c = _project(cell_features, W_cell, b_cell)

    def prep(idx):
        flat = idx.astype(jnp.int32).reshape(-1)
        return jnp.pad(flat, (0, nch_pad * idxw - flat.size))

    idx_d = prep(neigh_idx_drug)
    idx_g = prep(neigh_idx_gene)
    idx_c = prep(neigh_idx_cell)
    idx_s = jnp.pad(id_batch.astype(jnp.int32), (0, nself_pad - b))

    sc = _make_sc_gather_mean(k_fan, b_pad)
    agg_d, agg_g, agg_c, self_emb = sc(
        emb_d, emb_g, emb_c, idx_d, idx_g, idx_c, idx_s)

    return _combine(self_emb, agg_d, agg_g, agg_c, att, b)


# speedup vs baseline: 2.8457x; 1.1141x over previous
import functools

import jax
import jax.numpy as jnp
from jax import lax
from jax.experimental import pallas as pl
from jax.experimental.pallas import tpu as pltpu
from jax.experimental.pallas import tpu_sc as plsc

NW = 32
NS = 16
LANES = 16
D = 128
CHUNK_B = 8
SELF_B = 16
C0_ROWS = 560
C1_ROWS = 80
NBUF = 8
NPOOL = 16


def _project(x, w, b, bn=512):
    n, din = x.shape

    def body(x_ref, w_ref, b_ref, o_ref):
        o_ref[...] = (
            jnp.dot(x_ref[...].astype(jnp.bfloat16),
                    w_ref[...].astype(jnp.bfloat16),
                    preferred_element_type=jnp.float32)
            + b_ref[0:1, :]
        )

    return pl.pallas_call(
        body,
        grid=(pl.cdiv(n, bn),),
        in_specs=[
            pl.BlockSpec((bn, din), lambda i: (i, 0)),
            pl.BlockSpec((din, D), lambda i: (0, 0)),
            pl.BlockSpec((8, D), lambda i: (0, 0)),
        ],
        out_specs=pl.BlockSpec((bn, D), lambda i: (i, 0)),
        out_shape=jax.ShapeDtypeStruct((n, D), jnp.float32),
    )(x, w, jnp.pad(b.reshape(1, D), ((0, 7), (0, 0))))


def _make_sc_gather_mean(k_fan, b_pad):
    idxw = CHUNK_B * k_fan
    nch0 = C0_ROWS // CHUNK_B
    nch1 = C1_ROWS // CHUNK_B
    nself0 = C0_ROWS // SELF_B
    nself1 = C1_ROWS // SELF_B
    c0_total = NS * C0_ROWS
    mesh = plsc.VectorSubcoreMesh(core_axis_name="c", subcore_axis_name="s")
    out_sds = jax.ShapeDtypeStruct((b_pad, D), jnp.float32)

    @functools.partial(
        pl.kernel,
        out_type=(out_sds, out_sds, out_sds, out_sds),
        mesh=mesh,
        scratch_types=[
            pltpu.VMEM((nch0 * idxw,), jnp.int32),
            pltpu.VMEM((nself0 * SELF_B,), jnp.int32),
            pltpu.VMEM((NBUF, idxw, D), jnp.float32),
            pltpu.VMEM((NPOOL * CHUNK_B, D), jnp.float32),
            pltpu.SemaphoreType.DMA((NBUF,)),
            pltpu.SemaphoreType.DMA,
        ],
    )
    def sc_kernel(emb_d, emb_g, emb_c, idx_d, idx_g, idx_c, idx_s,
                  out_d, out_g, out_c, out_s,
                  idx_v, idx_sv, rows_v, pool_buf, sem, sem_out):
        c = lax.axis_index("c")
        s = lax.axis_index("s")
        is0 = c == 0
        nchunks = jnp.where(is0, nch0, nch1)
        nself = jnp.where(is0, nself0, nself1)
        row_base = pl.multiple_of(
            jnp.where(is0, s * C0_ROWS, c0_total + s * C1_ROWS), 16)
        idx_base = pl.multiple_of(row_base * k_fan, 16)
        inv_k = jnp.float32(1.0 / k_fan)

        def tree(vs):
            while len(vs) > 1:
                vs = [a + b for a, b in zip(vs[::2], vs[1::2])] + (
                    [vs[-1]] if len(vs) % 2 else [])
            return vs[0]

        for tbl, idx_hbm, out_hbm in (
            (emb_d, idx_d, out_d),
            (emb_g, idx_g, out_g),
            (emb_c, idx_c, out_c),
        ):
            pltpu.sync_copy(
                idx_hbm.at[pl.ds(idx_base, nch0 * idxw)], idx_v)

            def idx_at(ci):
                return idx_v.at[pl.ds(pl.multiple_of(ci * idxw, 16), idxw)]

            for c0 in range(NBUF - 1):
                pltpu.async_copy(
                    tbl.at[idx_at(c0)], rows_v.at[c0], sem.at[c0])

            def chunk_body(ci, carry, tbl=tbl, out_hbm=out_hbm, idx_at=idx_at):
                p = lax.rem(ci, NBUF)
                q = lax.rem(ci + NBUF - 1, NBUF)
                o = lax.rem(ci, NPOOL)
                pltpu.make_async_copy(
                    tbl.at[idx_at(ci)], rows_v.at[p], sem.at[p]).wait()

                @pl.when(ci + NBUF - 1 < nchunks)
                def _():
                    pltpu.async_copy(
                        tbl.at[idx_at(ci + NBUF - 1)], rows_v.at[q],
                        sem.at[q])

                @pl.when(ci >= NPOOL)
                def _():
                    pltpu.make_async_copy(
                        pool_buf.at[pl.ds(0, CHUNK_B)],
                        out_hbm.at[pl.ds(row_base, CHUNK_B)],
                        sem_out).wait()

                for r in range(CHUNK_B):
                    orow = o * CHUNK_B + r
                    for g in range(D // LANES):
                        sl = pl.ds(g * LANES, LANES)
                        pool_buf[orow, sl] = tree(
                            [rows_v[p, r * k_fan + k, sl]
                             for k in range(k_fan)]) * inv_k
                pltpu.async_copy(
                    pool_buf.at[pl.ds(
                        pl.multiple_of(o * CHUNK_B, CHUNK_B), CHUNK_B)],
                    out_hbm.at[pl.ds(
                        pl.multiple_of(row_base + ci * CHUNK_B, CHUNK_B),
                        CHUNK_B)],
                    sem_out)
                return carry

            lax.fori_loop(0, nchunks, chunk_body, 0)

            def drain_body(_, carry, out_hbm=out_hbm):
                pltpu.make_async_copy(
                    pool_buf.at[pl.ds(0, CHUNK_B)],
                    out_hbm.at[pl.ds(row_base, CHUNK_B)],
                    sem_out).wait()
                return carry

            lax.fori_loop(0, jnp.minimum(nchunks, NPOOL), drain_body, 0)

        pltpu.sync_copy(
            idx_s.at[pl.ds(row_base, nself0 * SELF_B)], idx_sv)

        nslot = NPOOL * CHUNK_B // SELF_B

        def self_slot(j):
            return pl.ds(pl.multiple_of(lax.rem(j, nslot) * SELF_B, SELF_B),
                         SELF_B)

        def fire_write(j):
            pltpu.async_copy(
                pool_buf.at[self_slot(j)],
                out_s.at[pl.ds(
                    pl.multiple_of(row_base + j * SELF_B, SELF_B), SELF_B)],
                sem_out)

        def wait_gather(j):
            pltpu.make_async_copy(
                emb_d.at[idx_sv.at[pl.ds(
                    pl.multiple_of(j * SELF_B, SELF_B), SELF_B)]],
                pool_buf.at[self_slot(j)],
                sem.at[lax.rem(j, NBUF)]).wait()

        def self_body(j, carry):
            @pl.when(j >= nslot)
            def _():
                pltpu.make_async_copy(
                    pool_buf.at[pl.ds(0, SELF_B)],
                    out_s.at[pl.ds(row_base, SELF_B)],
                    sem_out).wait()

            pltpu.async_copy(
                emb_d.at[idx_sv.at[pl.ds(
                    pl.multiple_of(j * SELF_B, SELF_B), SELF_B)]],
                pool_buf.at[self_slot(j)],
                sem.at[lax.rem(j, NBUF)])

            @pl.when(j >= 3)
            def _():
                wait_gather(j - 3)
                fire_write(j - 3)
            return carry

        lax.fori_loop(0, nself, self_body, 0)

        def self_tail(t, carry):
            j = nself - 3 + t
            wait_gather(j)
            fire_write(j)
            return carry

        lax.fori_loop(0, 3, self_tail, 0)
        def self_drain(_, carry):
            pltpu.make_async_copy(
                pool_buf.at[pl.ds(0, SELF_B)],
                out_s.at[pl.ds(row_base, SELF_B)],
                sem_out).wait()
            return carry

        lax.fori_loop(0, jnp.minimum(nself, nslot), self_drain, 0)

    return sc_kernel


def _combine(s, d, g, c, att, n_out, bn=512):
    def body(s_ref, d_ref, g_ref, c_ref, a_ref, o_ref):
        sv = s_ref[...]
        dv = d_ref[...]
        gv = g_ref[...]
        cv = c_ref[...]
        a1 = a_ref[0:1, :]
        a2 = a_ref[1:2, :]
        s_half = jnp.sum(sv * a1, axis=1, keepdims=True)

        def score(x):
            t = s_half + jnp.sum(x * a2, axis=1, keepdims=True)
            return jnp.where(t >= 0, t, 0.2 * t)

        t0, t1, t2, t3 = score(sv), score(dv), score(gv), score(cv)
        m = jnp.maximum(jnp.maximum(t0, t1), jnp.maximum(t2, t3))
        e0 = jnp.exp(t0 - m)
        e1 = jnp.exp(t1 - m)
        e2 = jnp.exp(t2 - m)
        e3 = jnp.exp(t3 - m)
        denom = e0 + e1 + e2 + e3
        o_ref[...] = (e0 * sv + e1 * dv + e2 * gv + e3 * cv) / denom

    spec = pl.BlockSpec((bn, D), lambda i: (i, 0))
    return pl.pallas_call(
        body,
        grid=(pl.cdiv(n_out, bn),),
        in_specs=[spec, spec, spec, spec,
                  pl.BlockSpec((8, D), lambda i: (0, 0))],
        out_specs=spec,
        out_shape=jax.ShapeDtypeStruct((n_out, D), jnp.float32),
    )(s, d, g, c, jnp.pad(att.reshape(2, D), ((0, 6), (0, 0))))


def kernel(id_batch, neigh_idx_drug, neigh_idx_gene, neigh_idx_cell,
           drug_features, gene_features, cell_features,
           W_drug, b_drug, W_gene, b_gene, W_cell, b_cell, att):
    b = id_batch.shape[0]
    k_fan = neigh_idx_drug.shape[1]
    b_pad = NS * (C0_ROWS + C1_ROWS)
    idxw = CHUNK_B * k_fan
    nrows = b_pad // CHUNK_B
    nch_pad = NS * (C0_ROWS // CHUNK_B) + (NS - 1) * (
        C1_ROWS // CHUNK_B) + C0_ROWS // CHUNK_B
    nself_pad = NS * C0_ROWS + (NS - 1) * C1_ROWS + C0_ROWS

    emb_d = _project(drug_features, W_drug, b_drug)
    emb_g = _project(gene_features, W_gene, b_gene)
    emb_c = _project(cell_features, W_cell, b_cell)

    def prep(idx):
        flat = idx.astype(jnp.int32).reshape(-1)
        return jnp.pad(flat, (0, nch_pad * idxw - flat.size))

    idx_d = prep(neigh_idx_drug)
    idx_g = prep(neigh_idx_gene)
    idx_c = prep(neigh_idx_cell)
    idx_s = jnp.pad(id_batch.astype(jnp.int32), (0, nself_pad - b))

    sc = _make_sc_gather_mean(k_fan, b_pad)
    agg_d, agg_g, agg_c, self_emb = sc(
        emb_d, emb_g, emb_c, idx_d, idx_g, idx_c, idx_s)

    return _combine(self_emb, agg_d, agg_g, agg_c, att, b)
